# Initial kernel scaffold; baseline (speedup 1.0000x reference)
import jax
import jax.numpy as jnp
from jax.experimental import pallas as pl


def kernel(edge_feats, segment_ids, W, b):
    raise NotImplementedError("write your pallas kernel here")


# speedup vs baseline: 3.3904x; 3.3904x over previous
import functools

import jax
import jax.numpy as jnp
from jax import lax
from jax.experimental import pallas as pl
from jax.experimental.pallas import tpu as pltpu
from jax.experimental.pallas import tpu_sc as plsc

E = 320000
D = 128
G = 256
NW = 32
RPW = E // NW
C = 400
NCHUNK = RPW // C
NV = D // 16

_mesh = plsc.VectorSubcoreMesh(core_axis_name="c", subcore_axis_name="s")


@functools.partial(
    pl.kernel,
    mesh=_mesh,
    out_type=jax.ShapeDtypeStruct((NW, G, 2 * D), jnp.float32),
    scratch_types=[
        pltpu.VMEM((C, D), jnp.float32),
        pltpu.VMEM((C + 16,), jnp.int32),
        pltpu.VMEM((D,), jnp.float32),
        pltpu.VMEM((16,), jnp.float32),
        pltpu.VMEM((G, 2 * D), jnp.float32),
    ],
)
def _sc_partials(edge_hbm, ids_hbm, w_hbm, b_hbm, out_hbm,
                 chunk_v, ids_v, w_v, b_v, acc_v):
    wid = lax.axis_index("s") * 2 + lax.axis_index("c")
    base = wid * RPW

    pltpu.sync_copy(w_hbm, w_v)
    pltpu.sync_copy(b_hbm, b_v)

    zeros = jnp.zeros((16,), jnp.float32)
    ninf = jnp.full((16,), -jnp.inf, jnp.float32)

    def init_g(g, carry):
        for v in range(NV):
            acc_v[g, pl.ds(v * 16, 16)] = zeros
            acc_v[g, pl.ds(D + v * 16, 16)] = ninf
        return carry

    lax.fori_loop(0, G, init_g, 0)

    wregs = [w_v[pl.ds(v * 16, 16)] for v in range(NV)]
    bvec = b_v[...]
    lanes = lax.iota(jnp.int32, 16)
    perms = [jnp.bitwise_xor(lanes, k) for k in (1, 2, 4, 8)]

    def chunk_body(k, carry):
        row0 = base + k * C
        pltpu.sync_copy(edge_hbm.at[pl.ds(row0, C)], chunk_v)
        pltpu.sync_copy(ids_hbm.at[pl.ds(row0, C)], ids_v.at[pl.ds(0, C)])

        def row_body(i, rcarry):
            xs = [chunk_v[i, pl.ds(v * 16, 16)] for v in range(NV)]
            p = xs[0] * wregs[0]
            for v in range(1, NV):
                p = p + xs[v] * wregs[v]
            for pm in perms:
                p = p + p.at[pm].get(mode="promise_in_bounds")
            sv = p + bvec
            e = jnp.exp(2.0 * sv)
            wt = 1.0 - 2.0 / (e + 1.0)
            seg = ids_v[pl.ds(i, 16)][0]
            for v in range(NV):
                cs = acc_v[seg, pl.ds(v * 16, 16)]
                acc_v[seg, pl.ds(v * 16, 16)] = cs + xs[v] * wt
                cm = acc_v[seg, pl.ds(D + v * 16, 16)]
                acc_v[seg, pl.ds(D + v * 16, 16)] = jnp.maximum(cm, xs[v])
            return rcarry

        lax.fori_loop(0, C, row_body, 0)
        return carry

    lax.fori_loop(0, NCHUNK, chunk_body, 0)
    pltpu.sync_copy(acc_v, out_hbm.at[wid])


def _combine_body(p_ref, o_ref):
    p = p_ref[...]
    o_ref[:, :D] = jnp.sum(p[:, :, :D], axis=0)
    o_ref[:, D:] = jnp.max(p[:, :, D:], axis=0)


_combine = pl.pallas_call(
    _combine_body,
    out_shape=jax.ShapeDtypeStruct((G, 2 * D), jnp.float32),
)


def kernel(edge_feats, segment_ids, W, b):
    ids = segment_ids.astype(jnp.int32)
    wf = W.reshape(D)
    b16 = jnp.full((16,), b[0], jnp.float32)
    partial = _sc_partials(edge_feats, ids, wf, b16)
    return _combine(partial)


# speedup vs baseline: 6.6844x; 1.9716x over previous
import functools

import jax
import jax.numpy as jnp
from jax import lax
from jax.experimental import pallas as pl
from jax.experimental.pallas import tpu as pltpu
from jax.experimental.pallas import tpu_sc as plsc

E = 320000
D = 128
G = 256
NW = 32
RPW = E // NW
C = 80
NCHUNK = RPW // C
NGRP = C // 16
NV = D // 16

_mesh = plsc.VectorSubcoreMesh(core_axis_name="c", subcore_axis_name="s")


@functools.partial(
    pl.kernel,
    mesh=_mesh,
    out_type=jax.ShapeDtypeStruct((NW, G, 2 * D), jnp.float32),
    scratch_types=[
        pltpu.VMEM((2, C, D), jnp.float32),
        pltpu.VMEM((2, C), jnp.int32),
        pltpu.VMEM((D,), jnp.float32),
        pltpu.VMEM((16,), jnp.float32),
        pltpu.VMEM((G, 2 * D), jnp.float32),
        pltpu.SemaphoreType.DMA((2,)),
    ],
)
def _sc_partials(edge_hbm, ids_hbm, w_hbm, b_hbm, out_hbm,
                 chunk_v, ids_v, w_v, b_v, acc_v, sem):
    wid = lax.axis_index("s") * 2 + lax.axis_index("c")
    base = wid * RPW

    pltpu.sync_copy(w_hbm, w_v)
    pltpu.sync_copy(b_hbm, b_v)

    zeros = jnp.zeros((16,), jnp.float32)
    ninf = jnp.full((16,), -jnp.inf, jnp.float32)

    def init_g(g, carry):
        for v in range(NV):
            acc_v[g, pl.ds(v * 16, 16)] = zeros
            acc_v[g, pl.ds(D + v * 16, 16)] = ninf
        return carry

    lax.fori_loop(0, G, init_g, 0)

    wregs = [w_v[pl.ds(v * 16, 16)] for v in range(NV)]
    bvec = b_v[...]
    lanes = lax.iota(jnp.int32, 16)
    perms = [jnp.bitwise_xor(lanes, k) for k in (1, 2, 4, 8)]

    def issue(k, par):
        row0 = base + k * C
        pltpu.async_copy(edge_hbm.at[pl.ds(row0, C)], chunk_v.at[par],
                         sem.at[par])
        pltpu.async_copy(ids_hbm.at[pl.ds(row0, C)], ids_v.at[par],
                         sem.at[par])

    def drain(k, par):
        row0 = base + k * C
        pltpu.make_async_copy(edge_hbm.at[pl.ds(row0, C)], chunk_v.at[par],
                              sem.at[par]).wait()
        pltpu.make_async_copy(ids_hbm.at[pl.ds(row0, C)], ids_v.at[par],
                              sem.at[par]).wait()

    def row_weight(xs):
        p = xs[0] * wregs[0]
        for v in range(1, NV):
            p = p + xs[v] * wregs[v]
        for pm in perms:
            p = p + p.at[pm].get(mode="promise_in_bounds")
        sv = p + bvec
        e = jnp.exp(2.0 * sv)
        return 1.0 - 2.0 / (e + 1.0)

    def load_row(par, i):
        return [chunk_v[par, i, pl.ds(v * 16, 16)] for v in range(NV)]

    issue(0, 0)

    def chunk_body(k, carry):
        par = lax.rem(k, 2)

        @pl.when(k + 1 < NCHUNK)
        def _prefetch():
            issue(k + 1, 1 - par)

        drain(k, par)

        def group_body(g, gcarry):
            idvec = ids_v[par, pl.ds(g * 16, 16)]
            seg0 = idvec[0]
            uniform = seg0 == idvec[15]
            i0 = g * 16

            @pl.when(uniform)
            def _fast():
                s = [zeros] * NV
                m = [ninf] * NV
                for j in range(16):
                    xs = load_row(par, i0 + j)
                    wt = row_weight(xs)
                    for v in range(NV):
                        s[v] = s[v] + xs[v] * wt
                        m[v] = jnp.maximum(m[v], xs[v])
                for v in range(NV):
                    cs = acc_v[seg0, pl.ds(v * 16, 16)]
                    acc_v[seg0, pl.ds(v * 16, 16)] = cs + s[v]
                    cm = acc_v[seg0, pl.ds(D + v * 16, 16)]
                    acc_v[seg0, pl.ds(D + v * 16, 16)] = jnp.maximum(cm, m[v])

            @pl.when(jnp.logical_not(uniform))
            def _slow():
                for j in range(16):
                    xs = load_row(par, i0 + j)
                    wt = row_weight(xs)
                    seg = idvec[j]
                    for v in range(NV):
                        cs = acc_v[seg, pl.ds(v * 16, 16)]
                        acc_v[seg, pl.ds(v * 16, 16)] = cs + xs[v] * wt
                        cm = acc_v[seg, pl.ds(D + v * 16, 16)]
                        acc_v[seg, pl.ds(D + v * 16, 16)] = \
                            jnp.maximum(cm, xs[v])

            return gcarry

        lax.fori_loop(0, NGRP, group_body, 0)
        return carry

    lax.fori_loop(0, NCHUNK, chunk_body, 0)
    pltpu.sync_copy(acc_v, out_hbm.at[wid])


def _combine_body(p_ref, o_ref):
    p = p_ref[...]
    o_ref[:, :D] = jnp.sum(p[:, :, :D], axis=0)
    o_ref[:, D:] = jnp.max(p[:, :, D:], axis=0)


_combine = pl.pallas_call(
    _combine_body,
    out_shape=jax.ShapeDtypeStruct((G, 2 * D), jnp.float32),
)


def kernel(edge_feats, segment_ids, W, b):
    ids = segment_ids.astype(jnp.int32)
    wf = W.reshape(D)
    b16 = jnp.full((16,), b[0], jnp.float32)
    partial = _sc_partials(edge_feats, ids, wf, b16)
    return _combine(partial)


# speedup vs baseline: 11.4525x; 1.7133x over previous
import functools

import jax
import jax.numpy as jnp
from jax import lax
from jax.experimental import pallas as pl
from jax.experimental.pallas import tpu as pltpu
from jax.experimental.pallas import tpu_sc as plsc

E = 320000
D = 128
G = 256
NW = 32
RPW = E // NW
C = 80
NCHUNK = RPW // C
NGRP = C // 16
NV = D // 16

_mesh = plsc.VectorSubcoreMesh(core_axis_name="c", subcore_axis_name="s")


@functools.partial(
    pl.kernel,
    mesh=_mesh,
    out_type=jax.ShapeDtypeStruct((NW, G, 2 * D), jnp.float32),
    scratch_types=[
        pltpu.VMEM((2, C, D), jnp.float32),
        pltpu.VMEM((2, C), jnp.int32),
        pltpu.VMEM((D,), jnp.float32),
        pltpu.VMEM((16,), jnp.float32),
        pltpu.VMEM((G, 2 * D), jnp.float32),
        pltpu.SemaphoreType.DMA((2,)),
    ],
)
def _sc_partials(edge_hbm, ids_hbm, w_hbm, b_hbm, out_hbm,
                 chunk_v, ids_v, w_v, b_v, acc_v, sem):
    wid = lax.axis_index("s") * 2 + lax.axis_index("c")
    base = wid * RPW

    pltpu.sync_copy(w_hbm, w_v)
    pltpu.sync_copy(b_hbm, b_v)

    zeros = jnp.zeros((16,), jnp.float32)
    ninf = jnp.full((16,), -jnp.inf, jnp.float32)

    def init_g(g, carry):
        for v in range(NV):
            acc_v[g, pl.ds(v * 16, 16)] = zeros
            acc_v[g, pl.ds(D + v * 16, 16)] = ninf
        return carry

    lax.fori_loop(0, G, init_g, 0)

    wregs = [w_v[pl.ds(v * 16, 16)] for v in range(NV)]
    bvec = b_v[...]
    lanes = lax.iota(jnp.int32, 16)
    perms_by_k = {k: jnp.bitwise_xor(lanes, k) for k in (1, 2, 4, 8)}

    def issue(k, par):
        row0 = base + k * C
        pltpu.async_copy(edge_hbm.at[pl.ds(row0, C)], chunk_v.at[par],
                         sem.at[par])
        pltpu.async_copy(ids_hbm.at[pl.ds(row0, C)], ids_v.at[par],
                         sem.at[par])

    def drain(k, par):
        row0 = base + k * C
        pltpu.make_async_copy(edge_hbm.at[pl.ds(row0, C)], chunk_v.at[par],
                              sem.at[par]).wait()
        pltpu.make_async_copy(ids_hbm.at[pl.ds(row0, C)], ids_v.at[par],
                              sem.at[par]).wait()

    masks = {k: (lanes & k) == 0 for k in (8, 4)}

    def fold(a, k):
        return a + a.at[perms_by_k[k]].get(mode="promise_in_bounds")

    def combine(a, b, k):
        return jnp.where(masks[k], fold(a, k), fold(b, k))

    def row_dot(xs):
        p = xs[0] * wregs[0]
        for v in range(1, NV):
            p = p + xs[v] * wregs[v]
        return p

    def tanh_vec(sv):
        e = jnp.exp(2.0 * sv)
        return 1.0 - 2.0 / (e + 1.0)

    def quad_weights(ps):
        t = combine(combine(ps[0], ps[1], 8), combine(ps[2], ps[3], 8), 4)
        t = t + t.at[perms_by_k[2]].get(mode="promise_in_bounds")
        t = t + t.at[perms_by_k[1]].get(mode="promise_in_bounds")
        return tanh_vec(t + bvec)

    def bcast(vec, lane):
        idx = jnp.full((16,), lane, jnp.int32)
        return vec.at[idx].get(mode="promise_in_bounds")

    QPOS = (0, 8, 4, 12)

    def row_weight(xs):
        p = row_dot(xs)
        for pm in (perms_by_k[1], perms_by_k[2], perms_by_k[4],
                   perms_by_k[8]):
            p = p + p.at[pm].get(mode="promise_in_bounds")
        return tanh_vec(p + bvec)

    def load_row(par, i):
        return [chunk_v[par, i, pl.ds(v * 16, 16)] for v in range(NV)]

    issue(0, 0)

    def chunk_body(k, carry):
        par = lax.rem(k, 2)

        @pl.when(k + 1 < NCHUNK)
        def _prefetch():
            issue(k + 1, 1 - par)

        drain(k, par)

        def group_body(g, gcarry):
            idvec = ids_v[par, pl.ds(g * 16, 16)]
            seg0 = idvec[0]
            uniform = seg0 == idvec[15]
            i0 = g * 16

            @pl.when(uniform)
            def _fast():
                s = [zeros] * NV
                m = [ninf] * NV
                for q in range(4):
                    xq = [load_row(par, i0 + 4 * q + r) for r in range(4)]
                    wtv = quad_weights([row_dot(xs) for xs in xq])
                    for r in range(4):
                        wt = bcast(wtv, QPOS[r])
                        for v in range(NV):
                            s[v] = s[v] + xq[r][v] * wt
                            m[v] = jnp.maximum(m[v], xq[r][v])
                for v in range(NV):
                    cs = acc_v[seg0, pl.ds(v * 16, 16)]
                    acc_v[seg0, pl.ds(v * 16, 16)] = cs + s[v]
                    cm = acc_v[seg0, pl.ds(D + v * 16, 16)]
                    acc_v[seg0, pl.ds(D + v * 16, 16)] = jnp.maximum(cm, m[v])

            @pl.when(jnp.logical_not(uniform))
            def _slow():
                for j in range(16):
                    xs = load_row(par, i0 + j)
                    wt = row_weight(xs)
                    seg = idvec[j]
                    for v in range(NV):
                        cs = acc_v[seg, pl.ds(v * 16, 16)]
                        acc_v[seg, pl.ds(v * 16, 16)] = cs + xs[v] * wt
                        cm = acc_v[seg, pl.ds(D + v * 16, 16)]
                        acc_v[seg, pl.ds(D + v * 16, 16)] = \
                            jnp.maximum(cm, xs[v])

            return gcarry

        lax.fori_loop(0, NGRP, group_body, 0)
        return carry

    lax.fori_loop(0, NCHUNK, chunk_body, 0)
    pltpu.sync_copy(acc_v, out_hbm.at[wid])


def _combine_body(p_ref, o_ref):
    p = p_ref[...]
    o_ref[:, :D] = jnp.sum(p[:, :, :D], axis=0)
    o_ref[:, D:] = jnp.max(p[:, :, D:], axis=0)


_combine = pl.pallas_call(
    _combine_body,
    out_shape=jax.ShapeDtypeStruct((G, 2 * D), jnp.float32),
)


def kernel(edge_feats, segment_ids, W, b):
    ids = segment_ids.astype(jnp.int32)
    wf = W.reshape(D)
    b16 = jnp.full((16,), b[0], jnp.float32)
    partial = _sc_partials(edge_feats, ids, wf, b16)
    return _combine(partial)
